# Initial kernel scaffold; baseline (speedup 1.0000x reference)
import jax
import jax.numpy as jnp
from jax.experimental import pallas as pl


def kernel(node_features, edge_src, edge_dst, edge_attr, edge_embedding, W1, b1, W2, b2):
    raise NotImplementedError("write your pallas kernel here")


# speedup vs baseline: 1.8987x; 1.8987x over previous
import functools

import jax
import jax.numpy as jnp
import numpy as np
from jax import lax
from jax.experimental import pallas as pl
from jax.experimental.pallas import tpu as pltpu
from jax.experimental.pallas import tpu_sc as plsc

N_NODES = 10000
N_EDGES = 160000
MUL_IN = 16
MUL_OUT = 16
DIM_EDGE_EMB = 64
HIDDEN = 16
WNUM = MUL_IN * MUL_OUT
NORM = 1.0 / np.sqrt(MUL_IN)

PACK = 128 // MUL_IN
N_COARSE = N_NODES // PACK
ACC_ROWS = 1280

NC = 2
NS = 16
NW = NC * NS

CH = 100
NCHUNK = N_EDGES // CH
CPT = NCHUNK // NW
CPT_SC = (NCHUNK // NC) // NS
ZROWS = ACC_ROWS // NS


@functools.cache
def _sc_kernels():
    mesh = plsc.VectorSubcoreMesh(core_axis_name="c", subcore_axis_name="s",
                                  num_cores=NC, num_subcores=NS)

    @functools.partial(
        pl.kernel,
        out_type=jax.ShapeDtypeStruct((NCHUNK, CH, 128), jnp.float32),
        mesh=mesh,
        scratch_types=[
            pltpu.VMEM((CPT, CH), jnp.int32),
            pltpu.VMEM((CH, 128), jnp.float32),
            pltpu.SemaphoreType.DMA,
        ],
    )
    def gather_k(table_hbm, src_hbm, out_hbm, idx_v, rows_v, sem):
        wid = lax.axis_index("c") * NS + lax.axis_index("s")
        c0 = wid * CPT
        pltpu.sync_copy(src_hbm.at[wid], idx_v)

        def body(j, _):
            pltpu.async_copy(table_hbm.at[idx_v.at[j]], rows_v, sem).wait()
            pltpu.sync_copy(rows_v, out_hbm.at[c0 + j])
            return _

        lax.fori_loop(0, CPT, body, None)

    @functools.partial(
        pl.kernel,
        out_type=jax.ShapeDtypeStruct((NC, ACC_ROWS, 128), jnp.float32),
        mesh=mesh,
        scratch_types=[
            pltpu.VMEM((CPT_SC, CH), jnp.int32),
            pltpu.VMEM((CH, 128), jnp.float32),
            pltpu.VMEM_SHARED((ACC_ROWS, 128), jnp.float32),
            pltpu.SemaphoreType.DMA,
        ],
    )
    def scatter_k(feat_hbm, dst_hbm, zeros_hbm, part_hbm, idx_v, feat_v, acc_sh, sem):
        cid = lax.axis_index("c")
        sid = lax.axis_index("s")
        r0 = sid * ZROWS
        pltpu.sync_copy(zeros_hbm.at[pl.ds(r0, ZROWS)], acc_sh.at[pl.ds(r0, ZROWS)])
        wid = cid * NS + sid
        c0 = wid * CPT_SC
        pltpu.sync_copy(dst_hbm.at[wid], idx_v)
        plsc.subcore_barrier()

        def body(j, _):
            pltpu.sync_copy(feat_hbm.at[c0 + j], feat_v)
            pltpu.sync_copy(feat_v, acc_sh.at[idx_v.at[j]], add=True)
            return _

        lax.fori_loop(0, CPT_SC, body, None)
        plsc.subcore_barrier()
        pltpu.sync_copy(acc_sh.at[pl.ds(r0, ZROWS)],
                        part_hbm.at[cid, pl.ds(r0, ZROWS)])

    return gather_k, scatter_k


BLK_E = 2000
GRID_E = N_EDGES // BLK_E


def _compute_body(y_ref, emb_ref, attr_ref, srcm_ref, dstm_ref,
                  w1_ref, b1_ref, w2_ref, b2_ref, sw_ref, r_ref, s_ref, rw_ref,
                  out_ref):
    lane_grp = lax.broadcasted_iota(jnp.int32, (BLK_E, 128), 1) // MUL_IN
    ysel = jnp.where(srcm_ref[...] == lane_grp, y_ref[...], 0.0)
    x1 = jnp.dot(ysel, sw_ref[...], preferred_element_type=jnp.float32)
    h = jnp.dot(emb_ref[...], w1_ref[...], preferred_element_type=jnp.float32)
    h = h + b1_ref[...]
    h = h * jax.nn.sigmoid(h)
    wt = jnp.dot(h, w2_ref[...], preferred_element_type=jnp.float32) + b2_ref[...]
    x1r = jnp.dot(x1, r_ref[...], preferred_element_type=jnp.float32)
    ef = jnp.dot(wt * x1r, s_ref[...], preferred_element_type=jnp.float32)
    ef = ef * (attr_ref[...] * NORM)
    efw = jnp.dot(ef, rw_ref[...], preferred_element_type=jnp.float32)
    out_ref[...] = jnp.where(dstm_ref[...] == lane_grp, efw, 0.0)


_compute_k = pl.pallas_call(
    _compute_body,
    grid=(GRID_E,),
    in_specs=[
        pl.BlockSpec((BLK_E, 128), lambda i: (i, 0)),
        pl.BlockSpec((BLK_E, DIM_EDGE_EMB), lambda i: (i, 0)),
        pl.BlockSpec((BLK_E, 1), lambda i: (i, 0)),
        pl.BlockSpec((BLK_E, 1), lambda i: (i, 0)),
        pl.BlockSpec((BLK_E, 1), lambda i: (i, 0)),
        pl.BlockSpec((DIM_EDGE_EMB, HIDDEN), lambda i: (0, 0)),
        pl.BlockSpec((1, HIDDEN), lambda i: (0, 0)),
        pl.BlockSpec((HIDDEN, WNUM), lambda i: (0, 0)),
        pl.BlockSpec((1, WNUM), lambda i: (0, 0)),
        pl.BlockSpec((128, MUL_IN), lambda i: (0, 0)),
        pl.BlockSpec((MUL_IN, WNUM), lambda i: (0, 0)),
        pl.BlockSpec((WNUM, MUL_OUT), lambda i: (0, 0)),
        pl.BlockSpec((MUL_OUT, 128), lambda i: (0, 0)),
    ],
    out_specs=pl.BlockSpec((BLK_E, 128), lambda i: (i, 0)),
    out_shape=jax.ShapeDtypeStruct((N_EDGES, 128), jnp.float32),
)


def _combine_body(p_ref, out_ref):
    out_ref[...] = p_ref[0, :N_COARSE] + p_ref[1, :N_COARSE]


_combine_k = pl.pallas_call(
    _combine_body,
    out_shape=jax.ShapeDtypeStruct((N_COARSE, 128), jnp.float32),
)


def kernel(node_features, edge_src, edge_dst, edge_attr, edge_embedding,
           W1, b1, W2, b2):
    gather_k, scatter_k = _sc_kernels()
    src = edge_src.astype(jnp.int32)
    dst = edge_dst.astype(jnp.int32)
    table128 = node_features.reshape(N_COARSE, 128)
    srcg = (src // PACK).reshape(NW, CPT, CH)
    dstg = (dst // PACK).reshape(NW, CPT_SC, CH)
    srcm = (src % PACK).reshape(N_EDGES, 1)
    dstm = (dst % PACK).reshape(N_EDGES, 1)

    y = gather_k(table128, srcg).reshape(N_EDGES, 128)

    k256 = jnp.arange(WNUM)
    k128 = jnp.arange(128)
    Sw = (k128[:, None] % MUL_IN == jnp.arange(MUL_IN)[None, :]).astype(jnp.float32)
    R = (k256[None, :] // MUL_OUT == jnp.arange(MUL_IN)[:, None]).astype(jnp.float32)
    S = (k256[:, None] % MUL_OUT == jnp.arange(MUL_OUT)[None, :]).astype(jnp.float32)
    Rw = (k128[None, :] % MUL_OUT == jnp.arange(MUL_OUT)[:, None]).astype(jnp.float32)

    efw = _compute_k(y, edge_embedding, edge_attr, srcm, dstm,
                     W1, b1.reshape(1, -1), W2, b2.reshape(1, -1), Sw, R, S, Rw)

    zeros = jnp.zeros((ACC_ROWS, 128), jnp.float32)
    partials = scatter_k(efw.reshape(NCHUNK, CH, 128), dstg, zeros)
    out = _combine_k(partials)
    return out.reshape(N_NODES, MUL_IN)


# speedup vs baseline: 2.0151x; 1.0613x over previous
import functools

import jax
import jax.numpy as jnp
import numpy as np
from jax import lax
from jax.experimental import pallas as pl
from jax.experimental.pallas import tpu as pltpu
from jax.experimental.pallas import tpu_sc as plsc

N_NODES = 10000
N_EDGES = 160000
MUL_IN = 16
MUL_OUT = 16
DIM_EDGE_EMB = 64
HIDDEN = 16
WNUM = MUL_IN * MUL_OUT
NORM = 1.0 / np.sqrt(MUL_IN)

PACK = 128 // MUL_IN
N_COARSE = N_NODES // PACK
ACC_ROWS = 1280

NC = 2
NS = 16
NW = NC * NS

CH = 100
NCHUNK = N_EDGES // CH
CPT = NCHUNK // NW
CPT_SC = (NCHUNK // NC) // NS
ZROWS = ACC_ROWS // NS
NBUF = 4


@functools.cache
def _sc_kernels():
    mesh = plsc.VectorSubcoreMesh(core_axis_name="c", subcore_axis_name="s",
                                  num_cores=NC, num_subcores=NS)

    @functools.partial(
        pl.kernel,
        out_type=jax.ShapeDtypeStruct((NCHUNK, CH, 128), jnp.float32),
        mesh=mesh,
        scratch_types=[
            pltpu.VMEM((CPT, CH), jnp.int32),
            pltpu.VMEM((NBUF, CH, 128), jnp.float32),
            pltpu.SemaphoreType.DMA,
            pltpu.SemaphoreType.DMA,
        ],
    )
    def gather_k(table_hbm, src_hbm, out_hbm, idx_v, rows_v, gsem, osem):
        wid = lax.axis_index("c") * NS + lax.axis_index("s")
        c0 = wid * CPT
        pltpu.sync_copy(src_hbm.at[wid], idx_v)

        for b in range(NBUF):
            pltpu.async_copy(table_hbm.at[idx_v.at[b]], rows_v.at[b], gsem)

        def body(j, _):
            buf = rows_v.at[lax.rem(j, NBUF)]
            pltpu.make_async_copy(table_hbm.at[idx_v.at[j]], buf, gsem).wait()
            ocp = pltpu.async_copy(buf, out_hbm.at[c0 + j], osem)

            @pl.when(j + NBUF < CPT)
            def _prefetch():
                ocp.wait()
                pltpu.async_copy(table_hbm.at[idx_v.at[j + NBUF]], buf, gsem)

            return _

        lax.fori_loop(0, CPT, body, None)
        for b in range(NBUF):
            pltpu.make_async_copy(rows_v.at[b], out_hbm.at[c0 + b], osem).wait()

    @functools.partial(
        pl.kernel,
        out_type=jax.ShapeDtypeStruct((NC, ACC_ROWS, 128), jnp.float32),
        mesh=mesh,
        scratch_types=[
            pltpu.VMEM((CPT_SC, CH), jnp.int32),
            pltpu.VMEM((NBUF, CH, 128), jnp.float32),
            pltpu.VMEM_SHARED((ACC_ROWS, 128), jnp.float32),
            pltpu.SemaphoreType.DMA,
            pltpu.SemaphoreType.DMA,
        ],
    )
    def scatter_k(feat_hbm, dst_hbm, zeros_hbm, part_hbm, idx_v, feat_v, acc_sh,
                  lsem, ssem):
        cid = lax.axis_index("c")
        sid = lax.axis_index("s")
        r0 = sid * ZROWS
        pltpu.sync_copy(zeros_hbm.at[pl.ds(r0, ZROWS)], acc_sh.at[pl.ds(r0, ZROWS)])
        wid = cid * NS + sid
        c0 = wid * CPT_SC
        pltpu.sync_copy(dst_hbm.at[wid], idx_v)
        plsc.subcore_barrier()

        for b in range(NBUF):
            pltpu.async_copy(feat_hbm.at[c0 + b], feat_v.at[b], lsem)

        def body(j, _):
            buf = feat_v.at[lax.rem(j, NBUF)]
            pltpu.make_async_copy(feat_hbm.at[c0 + j], buf, lsem).wait()
            scp = pltpu.async_copy(buf, acc_sh.at[idx_v.at[j]], ssem, add=True)

            @pl.when(j + NBUF < CPT_SC)
            def _prefetch():
                scp.wait()
                pltpu.async_copy(feat_hbm.at[c0 + j + NBUF], buf, lsem)

            return _

        lax.fori_loop(0, CPT_SC, body, None)
        for b in range(NBUF):
            pltpu.make_async_copy(feat_v.at[b], acc_sh.at[idx_v.at[b]], ssem).wait()
        plsc.subcore_barrier()
        pltpu.sync_copy(acc_sh.at[pl.ds(r0, ZROWS)],
                        part_hbm.at[cid, pl.ds(r0, ZROWS)])

    return gather_k, scatter_k


BLK_E = 2000
GRID_E = N_EDGES // BLK_E


def _compute_body(y_ref, emb_ref, attr_ref, srcm_ref, dstm_ref,
                  w1_ref, b1_ref, w2_ref, b2_ref, sw_ref, r_ref, s_ref, rw_ref,
                  out_ref):
    lane_grp = lax.broadcasted_iota(jnp.int32, (BLK_E, 128), 1) // MUL_IN
    ysel = jnp.where(srcm_ref[...] == lane_grp, y_ref[...], 0.0)
    x1 = jnp.dot(ysel, sw_ref[...], preferred_element_type=jnp.float32)
    h = jnp.dot(emb_ref[...], w1_ref[...], preferred_element_type=jnp.float32)
    h = h + b1_ref[...]
    h = h * jax.nn.sigmoid(h)
    wt = jnp.dot(h, w2_ref[...], preferred_element_type=jnp.float32) + b2_ref[...]
    x1r = jnp.dot(x1, r_ref[...], preferred_element_type=jnp.float32)
    ef = jnp.dot(wt * x1r, s_ref[...], preferred_element_type=jnp.float32)
    ef = ef * (attr_ref[...] * NORM)
    efw = jnp.dot(ef, rw_ref[...], preferred_element_type=jnp.float32)
    out_ref[...] = jnp.where(dstm_ref[...] == lane_grp, efw, 0.0)


_compute_k = pl.pallas_call(
    _compute_body,
    grid=(GRID_E,),
    in_specs=[
        pl.BlockSpec((BLK_E, 128), lambda i: (i, 0)),
        pl.BlockSpec((BLK_E, DIM_EDGE_EMB), lambda i: (i, 0)),
        pl.BlockSpec((BLK_E, 1), lambda i: (i, 0)),
        pl.BlockSpec((BLK_E, 1), lambda i: (i, 0)),
        pl.BlockSpec((BLK_E, 1), lambda i: (i, 0)),
        pl.BlockSpec((DIM_EDGE_EMB, HIDDEN), lambda i: (0, 0)),
        pl.BlockSpec((1, HIDDEN), lambda i: (0, 0)),
        pl.BlockSpec((HIDDEN, WNUM), lambda i: (0, 0)),
        pl.BlockSpec((1, WNUM), lambda i: (0, 0)),
        pl.BlockSpec((128, MUL_IN), lambda i: (0, 0)),
        pl.BlockSpec((MUL_IN, WNUM), lambda i: (0, 0)),
        pl.BlockSpec((WNUM, MUL_OUT), lambda i: (0, 0)),
        pl.BlockSpec((MUL_OUT, 128), lambda i: (0, 0)),
    ],
    out_specs=pl.BlockSpec((BLK_E, 128), lambda i: (i, 0)),
    out_shape=jax.ShapeDtypeStruct((N_EDGES, 128), jnp.float32),
)


def _combine_body(p_ref, out_ref):
    out_ref[...] = p_ref[0, :N_COARSE] + p_ref[1, :N_COARSE]


_combine_k = pl.pallas_call(
    _combine_body,
    out_shape=jax.ShapeDtypeStruct((N_COARSE, 128), jnp.float32),
)


def kernel(node_features, edge_src, edge_dst, edge_attr, edge_embedding,
           W1, b1, W2, b2):
    gather_k, scatter_k = _sc_kernels()
    src = edge_src.astype(jnp.int32)
    dst = edge_dst.astype(jnp.int32)
    table128 = node_features.reshape(N_COARSE, 128)
    srcg = (src // PACK).reshape(NW, CPT, CH)
    dstg = (dst // PACK).reshape(NW, CPT_SC, CH)
    srcm = (src % PACK).reshape(N_EDGES, 1)
    dstm = (dst % PACK).reshape(N_EDGES, 1)

    y = gather_k(table128, srcg).reshape(N_EDGES, 128)

    k256 = jnp.arange(WNUM)
    k128 = jnp.arange(128)
    Sw = (k128[:, None] % MUL_IN == jnp.arange(MUL_IN)[None, :]).astype(jnp.float32)
    R = (k256[None, :] // MUL_OUT == jnp.arange(MUL_IN)[:, None]).astype(jnp.float32)
    S = (k256[:, None] % MUL_OUT == jnp.arange(MUL_OUT)[None, :]).astype(jnp.float32)
    Rw = (k128[None, :] % MUL_OUT == jnp.arange(MUL_OUT)[:, None]).astype(jnp.float32)

    efw = _compute_k(y, edge_embedding, edge_attr, srcm, dstm,
                     W1, b1.reshape(1, -1), W2, b2.reshape(1, -1), Sw, R, S, Rw)

    zeros = jnp.zeros((ACC_ROWS, 128), jnp.float32)
    partials = scatter_k(efw.reshape(NCHUNK, CH, 128), dstg, zeros)
    out = _combine_k(partials)
    return out.reshape(N_NODES, MUL_IN)


# speedup vs baseline: 4.8442x; 2.4040x over previous
import functools

import jax
import jax.numpy as jnp
import numpy as np
from jax import lax
from jax.experimental import pallas as pl
from jax.experimental.pallas import tpu as pltpu
from jax.experimental.pallas import tpu_sc as plsc

N_NODES = 10000
N_EDGES = 160000
MUL_IN = 16
MUL_OUT = 16
DIM_EDGE_EMB = 64
HIDDEN = 16
WNUM = MUL_IN * MUL_OUT
NORM = 1.0 / np.sqrt(MUL_IN)

NC = 2
NS = 16
NW = NC * NS

CH = 40
NCHUNK = N_EDGES // CH
CPT = NCHUNK // NW
CPT_SC = (NCHUNK // NC) // NS
ACC_ROWS = 10240
ZROWS = ACC_ROWS // NS
NBUF = 4


@functools.cache
def _sc_kernels():
    mesh = plsc.VectorSubcoreMesh(core_axis_name="c", subcore_axis_name="s",
                                  num_cores=NC, num_subcores=NS)

    @functools.partial(
        pl.kernel,
        out_type=jax.ShapeDtypeStruct((N_EDGES, 128), jnp.float32),
        mesh=mesh,
        scratch_types=[
            pltpu.VMEM((CPT, CH), jnp.int32),
            pltpu.VMEM((NBUF, CH, 128), jnp.float32),
            pltpu.SemaphoreType.DMA,
            pltpu.SemaphoreType.DMA,
        ],
    )
    def gather_k(table_hbm, src_hbm, out_hbm, idx_v, rows_v, gsem, osem):
        wid = lax.axis_index("c") * NS + lax.axis_index("s")
        c0 = wid * CPT
        pltpu.sync_copy(src_hbm.at[wid], idx_v)

        for b in range(NBUF):
            pltpu.async_copy(table_hbm.at[idx_v.at[b]], rows_v.at[b], gsem)

        def body(j, _):
            buf = rows_v.at[lax.rem(j, NBUF)]
            pltpu.make_async_copy(table_hbm.at[idx_v.at[j]], buf, gsem).wait()
            ocp = pltpu.async_copy(buf, out_hbm.at[pl.ds((c0 + j) * CH, CH)], osem)

            @pl.when(j + NBUF < CPT)
            def _prefetch():
                ocp.wait()
                pltpu.async_copy(table_hbm.at[idx_v.at[j + NBUF]], buf, gsem)

            return _

        lax.fori_loop(0, CPT, body, None)
        for b in range(NBUF):
            pltpu.make_async_copy(rows_v.at[b],
                                  out_hbm.at[pl.ds(c0 * CH, CH)], osem).wait()

    @functools.partial(
        pl.kernel,
        out_type=jax.ShapeDtypeStruct((NC, ACC_ROWS, 128), jnp.float32),
        mesh=mesh,
        scratch_types=[
            pltpu.VMEM((CPT_SC, CH), jnp.int32),
            pltpu.VMEM((NBUF, CH, 128), jnp.float32),
            pltpu.VMEM_SHARED((ACC_ROWS, 128), jnp.float32),
            pltpu.SemaphoreType.DMA,
            pltpu.SemaphoreType.DMA,
        ],
    )
    def scatter_k(feat_hbm, dst_hbm, zeros_hbm, part_hbm, idx_v, feat_v, acc_sh,
                  lsem, ssem):
        cid = lax.axis_index("c")
        sid = lax.axis_index("s")
        r0 = sid * ZROWS
        pltpu.sync_copy(zeros_hbm, acc_sh.at[pl.ds(r0, ZROWS)])
        wid = cid * NS + sid
        c0 = wid * CPT_SC
        pltpu.sync_copy(dst_hbm.at[wid], idx_v)
        plsc.subcore_barrier()

        for b in range(NBUF):
            pltpu.async_copy(feat_hbm.at[pl.ds((c0 + b) * CH, CH)],
                             feat_v.at[b], lsem)

        def body(j, _):
            buf = feat_v.at[lax.rem(j, NBUF)]
            pltpu.make_async_copy(feat_hbm.at[pl.ds((c0 + j) * CH, CH)],
                                  buf, lsem).wait()
            scp = pltpu.async_copy(buf, acc_sh.at[idx_v.at[j]], ssem, add=True)

            @pl.when(j + NBUF < CPT_SC)
            def _prefetch():
                scp.wait()
                pltpu.async_copy(feat_hbm.at[pl.ds((c0 + j + NBUF) * CH, CH)],
                                 buf, lsem)

            return _

        lax.fori_loop(0, CPT_SC, body, None)
        for b in range(NBUF):
            pltpu.make_async_copy(feat_v.at[b], acc_sh.at[idx_v.at[b]], ssem).wait()
        plsc.subcore_barrier()
        pltpu.sync_copy(acc_sh.at[pl.ds(r0, ZROWS)],
                        part_hbm.at[cid, pl.ds(r0, ZROWS)])

    return gather_k, scatter_k


BLK_E = 3200
GRID_E = N_EDGES // BLK_E


def _compute_body(y_ref, embt_ref, attrt_ref, w1_ref, b1_ref, w2_ref, b2_ref,
                  r_ref, s_ref, p_ref, out_ref):
    x1 = y_ref[:, :MUL_IN]
    h = lax.dot_general(embt_ref[...], w1_ref[...], (((0,), (0,)), ((), ())),
                        preferred_element_type=jnp.float32)
    h = h + b1_ref[...]
    h = h * jax.nn.sigmoid(h)
    wt = jnp.dot(h, w2_ref[...], preferred_element_type=jnp.float32) + b2_ref[...]
    x1r = jnp.dot(x1, r_ref[...], preferred_element_type=jnp.float32)
    attr_col = lax.transpose(attrt_ref[...], (1, 0))
    ef = jnp.dot(wt * x1r, s_ref[...], preferred_element_type=jnp.float32)
    ef = ef * (attr_col * NORM)
    out_ref[...] = jnp.dot(ef, p_ref[...], preferred_element_type=jnp.float32)


_compute_k = pl.pallas_call(
    _compute_body,
    grid=(GRID_E,),
    in_specs=[
        pl.BlockSpec((BLK_E, 128), lambda i: (i, 0)),
        pl.BlockSpec((DIM_EDGE_EMB, BLK_E), lambda i: (0, i)),
        pl.BlockSpec((1, BLK_E), lambda i: (0, i)),
        pl.BlockSpec((DIM_EDGE_EMB, HIDDEN), lambda i: (0, 0)),
        pl.BlockSpec((1, HIDDEN), lambda i: (0, 0)),
        pl.BlockSpec((HIDDEN, WNUM), lambda i: (0, 0)),
        pl.BlockSpec((1, WNUM), lambda i: (0, 0)),
        pl.BlockSpec((MUL_IN, WNUM), lambda i: (0, 0)),
        pl.BlockSpec((WNUM, MUL_OUT), lambda i: (0, 0)),
        pl.BlockSpec((MUL_OUT, 128), lambda i: (0, 0)),
    ],
    out_specs=pl.BlockSpec((BLK_E, 128), lambda i: (i, 0)),
    out_shape=jax.ShapeDtypeStruct((N_EDGES, 128), jnp.float32),
)


BLK_N = 1000
GRID_N = N_NODES // BLK_N


def _combine_body(p_ref, out_ref):
    out_ref[...] = p_ref[0, :, :MUL_OUT] + p_ref[1, :, :MUL_OUT]


_combine_k = pl.pallas_call(
    _combine_body,
    grid=(GRID_N,),
    in_specs=[pl.BlockSpec((NC, BLK_N, 128), lambda i: (0, i, 0))],
    out_specs=pl.BlockSpec((BLK_N, MUL_OUT), lambda i: (i, 0)),
    out_shape=jax.ShapeDtypeStruct((N_NODES, MUL_OUT), jnp.float32),
)


def kernel(node_features, edge_src, edge_dst, edge_attr, edge_embedding,
           W1, b1, W2, b2):
    gather_k, scatter_k = _sc_kernels()
    src3 = edge_src.astype(jnp.int32).reshape(NW, CPT, CH)
    dst3 = edge_dst.astype(jnp.int32).reshape(NW, CPT_SC, CH)
    table = jnp.tile(node_features, (1, 128 // MUL_IN))

    y = gather_k(table, src3)

    k256 = jnp.arange(WNUM)
    R = (k256[None, :] // MUL_OUT == jnp.arange(MUL_IN)[:, None]).astype(jnp.float32)
    S = (k256[:, None] % MUL_OUT == jnp.arange(MUL_OUT)[None, :]).astype(jnp.float32)
    P = (jnp.arange(128)[None, :] == jnp.arange(MUL_OUT)[:, None]).astype(jnp.float32)

    embt = jnp.swapaxes(edge_embedding, 0, 1)
    attrt = jnp.swapaxes(edge_attr, 0, 1)
    efw = _compute_k(y, embt, attrt, W1, b1.reshape(1, -1), W2, b2.reshape(1, -1),
                     R, S, P)

    zeros = jnp.zeros((ZROWS, 128), jnp.float32)
    partials = scatter_k(efw, dst3, zeros)
    return _combine_k(partials)
